# Initial kernel scaffold; baseline (speedup 1.0000x reference)
import jax
import jax.numpy as jnp
from jax.experimental import pallas as pl


def kernel(pos, edge_index):
    raise NotImplementedError("write your pallas kernel here")


# speedup vs baseline: 6.4346x; 6.4346x over previous
import functools

import numpy as np

import jax
import jax.numpy as jnp
from jax import lax
from jax.experimental import pallas as pl
from jax.experimental.pallas import tpu as pltpu
from jax.experimental.pallas import tpu_sc as plsc

_RADIUS = 0.015
_INV_R = 1.0 / _RADIUS
_HIST = 5
_ENV = 3

_NF = _HIST * _ENV + 2 * _ENV
_NODE_G = np.zeros((_ENV, _NF), np.float32)
_NODE_B = np.zeros((1, _NF), np.float32)
for _c in range(_ENV):
    _NODE_G[_c, _HIST * _ENV + 2 * _c] = 1.0
    _NODE_G[_c, _HIST * _ENV + 2 * _c + 1] = 1.0
    _NODE_B[0, _HIST * _ENV + 2 * _c + 1] = 1.0

_NW = 32
_GB = 128
_SB = 2048
_NSUB = _SB // _GB


def _perm(v, idx):
    return lax.gather(
        v, idx[:, None],
        lax.GatherDimensionNumbers(
            offset_dims=(), collapsed_slice_dims=(0,), start_index_map=(0,)),
        slice_sizes=(1,),
        mode=lax.GatherScatterMode.PROMISE_IN_BOUNDS)


def _rsqrt_newton(x):
    i = lax.bitcast_convert_type(x, jnp.int32)
    i = jnp.int32(0x5F3759DF) - lax.shift_right_logical(i, 1)
    y = lax.bitcast_convert_type(i, jnp.float32)
    xh = x * 0.5
    for _ in range(3):
        y = y * (1.5 - xh * y * y)
    return y


@functools.lru_cache(maxsize=None)
def _edge_kernel(n_nodes, n_pad):
    nsb = n_pad // (_NW * _SB)
    assert nsb % 2 == 0
    mesh = plsc.VectorSubcoreMesh(core_axis_name="c", subcore_axis_name="s")

    @functools.partial(
        pl.kernel,
        out_type=jax.ShapeDtypeStruct((n_pad * 4,), jnp.float32),
        mesh=mesh,
        scratch_types=[pltpu.VMEM((_NSUB, _GB), jnp.int32)] * 4
          + [pltpu.VMEM((_SB,), jnp.float32)] * 12
          + [pltpu.VMEM((_SB * 4,), jnp.float32)] * 2
          + [pltpu.SemaphoreType.DMA] * 4,
    )
    def ek(px, py, pz, src2d, dst2d, out,
           sidx0, didx0, sidx1, didx1,
           sx0, sy0, sz0, tx0, ty0, tz0,
           sx1, sy1, sz1, tx1, ty1, tz1,
           obuf0, obuf1, semg0, semg1, semo0, semo1):
        sid = lax.axis_index("s")
        cid = lax.axis_index("c")
        wid = sid * 2 + cid

        iota = lax.iota(jnp.int32, 16)
        lane_c = lax.rem(iota, 4)
        m0 = lane_c == 0
        m1 = lane_c == 1
        m2 = lane_c == 2
        pidx0 = lax.div(iota, 4)

        sets = (
            (sidx0, didx0, (sx0, sy0, sz0), (tx0, ty0, tz0), obuf0, semg0, semo0),
            (sidx1, didx1, (sx1, sy1, sz1), (tx1, ty1, tz1), obuf1, semg1, semo1),
        )
        sh = (px, py, pz)

        def fire(b, s):
            sidx, didx, sbufs, tbufs, _, semg, _ = sets[s]
            row0 = (wid * nsb + b) * _NSUB
            pltpu.sync_copy(src2d.at[pl.ds(row0, _NSUB)], sidx)
            pltpu.sync_copy(dst2d.at[pl.ds(row0, _NSUB)], didx)
            for j in range(_NSUB):
                sl = pl.ds(j * _GB, _GB)
                for c in range(3):
                    pltpu.async_copy(sh[c].at[sidx.at[j]], sbufs[c].at[sl], semg)
                    pltpu.async_copy(sh[c].at[didx.at[j]], tbufs[c].at[sl], semg)

        def wait_gathers(s):
            _, _, sbufs, tbufs, _, semg, _ = sets[s]
            for buf in sbufs + tbufs:
                pltpu.make_async_copy(px.at[pl.ds(0, _SB)], buf, semg).wait()

        def compute_write(b, s):
            _, _, (sxv, syv, szv), (txv, tyv, tzv), obuf, _, semo = sets[s]

            @pl.when(b >= 2)
            def _drain():
                pltpu.make_async_copy(
                    out.at[pl.ds(0, _SB * 4)], obuf, semo).wait()

            def compute_body(e, carry2):
                sl = pl.ds(e * 16, 16)
                dx = (sxv[sl] - txv[sl]) * _INV_R
                dy = (syv[sl] - tyv[sl]) * _INV_R
                dz = (szv[sl] - tzv[sl]) * _INV_R
                d2 = dx * dx + dy * dy + dz * dz + 1e-12
                dist = d2 * _rsqrt_newton(d2)
                for k in range(4):
                    pk = pidx0 + (4 * k)
                    ok = jnp.where(
                        m0, _perm(dx, pk),
                        jnp.where(m1, _perm(dy, pk),
                                  jnp.where(m2, _perm(dz, pk),
                                            _perm(dist, pk))))
                    obuf[pl.ds(e * 64 + k * 16, 16)] = ok
                return carry2

            lax.fori_loop(0, _SB // 16, compute_body, 0)
            pltpu.async_copy(
                obuf, out.at[pl.ds((wid * nsb + b) * _SB * 4, _SB * 4)], semo)

        fire(0, 0)

        def two_blocks(bb, carry):
            b0 = 2 * bb
            b1 = 2 * bb + 1
            wait_gathers(0)
            fire(b1, 1)
            compute_write(b0, 0)
            wait_gathers(1)

            @pl.when(b1 + 1 < nsb)
            def _next():
                fire(b1 + 1, 0)

            compute_write(b1, 1)
            return carry

        lax.fori_loop(0, nsb // 2, two_blocks, 0)

        pltpu.make_async_copy(out.at[pl.ds(0, _SB * 4)], obuf0, semo0).wait()
        pltpu.make_async_copy(out.at[pl.ds(0, _SB * 4)], obuf1, semo1).wait()

    return ek


def _node_body(pos_ref, g_ref, b_ref, out_ref):
    p = pos_ref[...]
    v = lax.dot_general(p, g_ref[...], (((1,), (0,)), ((), ())),
                        precision=lax.Precision.HIGHEST,
                        preferred_element_type=jnp.float32)
    out_ref[...] = jnp.clip((v - b_ref[...]) * _INV_R, -1.0, 1.0)


@functools.lru_cache(maxsize=None)
def _node_kernel(n):
    nb = 10000
    assert n % nb == 0
    return pl.pallas_call(
        _node_body,
        grid=(n // nb,),
        in_specs=[
            pl.BlockSpec((nb, _ENV), lambda i: (i, 0)),
            pl.BlockSpec((_ENV, _NF), lambda i: (0, 0)),
            pl.BlockSpec((1, _NF), lambda i: (0, 0)),
        ],
        out_specs=pl.BlockSpec((nb, 2 * _ENV + _HIST * _ENV), lambda i: (i, 0)),
        out_shape=jax.ShapeDtypeStruct((n, 2 * _ENV + _HIST * _ENV), jnp.float32),
    )


def kernel(pos, edge_index):
    n = pos.shape[0]
    m = edge_index.shape[1]
    chunk = _NW * _SB * 2
    n_pad = ((m + chunk - 1) // chunk) * chunk

    posf = pos.astype(jnp.float32)
    px = posf[:, 0]
    py = posf[:, 1]
    pz = posf[:, 2]
    src = jnp.pad(edge_index[0], (0, n_pad - m)).reshape(n_pad // _GB, _GB)
    dst = jnp.pad(edge_index[1], (0, n_pad - m)).reshape(n_pad // _GB, _GB)

    ef_flat = _edge_kernel(n, n_pad)(px, py, pz, src, dst)
    edge_feature = ef_flat.reshape(n_pad, 4)[:m]
    node_feature = _node_kernel(n)(
        posf, jnp.asarray(_NODE_G), jnp.asarray(_NODE_B))
    return node_feature, edge_feature


# speedup vs baseline: 8.2594x; 1.2836x over previous
import functools

import numpy as np

import jax
import jax.numpy as jnp
from jax import lax
from jax.experimental import pallas as pl
from jax.experimental.pallas import tpu as pltpu
from jax.experimental.pallas import tpu_sc as plsc

_RADIUS = 0.015
_INV_R = 1.0 / _RADIUS
_HIST = 5
_ENV = 3

_NF = _HIST * _ENV + 2 * _ENV
_NODE_G = np.zeros((_ENV, _NF), np.float32)
_NODE_B = np.zeros((1, _NF), np.float32)
for _c in range(_ENV):
    _NODE_G[_c, _HIST * _ENV + 2 * _c] = 1.0
    _NODE_G[_c, _HIST * _ENV + 2 * _c + 1] = 1.0
    _NODE_B[0, _HIST * _ENV + 2 * _c + 1] = 1.0

_NW = 32
_GBI = 512
_SB = 2048
_NG = _SB * 4 // _GBI


def _perm(v, idx):
    return lax.gather(
        v, idx[:, None],
        lax.GatherDimensionNumbers(
            offset_dims=(), collapsed_slice_dims=(0,), start_index_map=(0,)),
        slice_sizes=(1,),
        mode=lax.GatherScatterMode.PROMISE_IN_BOUNDS)


def _rsqrt_newton(x):
    i = lax.bitcast_convert_type(x, jnp.int32)
    i = jnp.int32(0x5F3759DF) - lax.shift_right_logical(i, 1)
    y = lax.bitcast_convert_type(i, jnp.float32)
    xh = x * 0.5
    for _ in range(3):
        y = y * (1.5 - xh * y * y)
    return y


@functools.lru_cache(maxsize=None)
def _edge_kernel(n_nodes, n_pad):
    nsb = n_pad // (_NW * _SB)
    assert nsb % 2 == 0
    mesh = plsc.VectorSubcoreMesh(core_axis_name="c", subcore_axis_name="s")

    @functools.partial(
        pl.kernel,
        out_type=jax.ShapeDtypeStruct((n_pad * 4,), jnp.float32),
        mesh=mesh,
        scratch_types=[pltpu.VMEM_SHARED((n_nodes * 4,), jnp.float32)]
          + [pltpu.VMEM((_SB,), jnp.int32)] * 4
          + [pltpu.VMEM((_SB * 4,), jnp.int32)] * 4
          + [pltpu.VMEM((_SB * 4,), jnp.float32)] * 4
          + [pltpu.VMEM((_SB * 4,), jnp.float32)] * 2
          + [pltpu.SemaphoreType.DMA] * 4,
    )
    def ek(pos4, srcf, dstf, out,
           sh4,
           sraw0, draw0, sraw1, draw1,
           si40, di40, si41, di41,
           srow0, drow0, srow1, drow1,
           obuf0, obuf1, semg0, semg1, semo0, semo1):
        sid = lax.axis_index("s")
        cid = lax.axis_index("c")
        wid = sid * 2 + cid

        @pl.when(sid == 0)
        def _stage():
            pltpu.sync_copy(pos4, sh4)

        plsc.subcore_barrier()

        iota = lax.iota(jnp.int32, 16)
        lane_c = lax.rem(iota, 4)
        rep4 = lax.div(iota, 4)
        swap1 = iota ^ 1
        swap2 = iota ^ 2
        m3 = lane_c == 3

        sets = (
            (sraw0, draw0, si40, di40, srow0, drow0, obuf0, semg0, semo0),
            (sraw1, draw1, si41, di41, srow1, drow1, obuf1, semg1, semo1),
        )

        def build_idx4(raw, idx4):
            def bb(v, carry):
                rv = raw[pl.ds(v * 16, 16)]
                for q in range(4):
                    pk = rep4 + (4 * q)
                    idx4[pl.ds(v * 64 + q * 16, 16)] = (
                        _perm(rv, pk) * 4 + lane_c)
                return carry
            lax.fori_loop(0, _SB // 16, bb, 0)

        def fire(b, s):
            sraw, draw, si4, di4, srow, drow, _, semg, _ = sets[s]
            e0 = (wid * nsb + b) * _SB
            pltpu.sync_copy(srcf.at[pl.ds(e0, _SB)], sraw)
            pltpu.sync_copy(dstf.at[pl.ds(e0, _SB)], draw)
            build_idx4(sraw, si4)
            build_idx4(draw, di4)
            for g in range(_NG):
                sl = pl.ds(g * _GBI, _GBI)
                pltpu.async_copy(sh4.at[si4.at[sl]], srow.at[sl], semg)
                pltpu.async_copy(sh4.at[di4.at[sl]], drow.at[sl], semg)

        def wait_gathers(s):
            srow, drow, semg = sets[s][4], sets[s][5], sets[s][7]
            pltpu.make_async_copy(out.at[pl.ds(0, _SB * 4)], srow, semg).wait()
            pltpu.make_async_copy(out.at[pl.ds(0, _SB * 4)], drow, semg).wait()

        def compute_write(b, s):
            _, _, _, _, srow, drow, obuf, _, semo = sets[s]

            @pl.when(b >= 2)
            def _drain():
                pltpu.make_async_copy(
                    out.at[pl.ds(0, _SB * 4)], obuf, semo).wait()

            def compute_body(v, carry2):
                sl = pl.ds(v * 16, 16)
                diff = (srow[sl] - drow[sl]) * _INV_R
                sq = diff * diff
                s1 = sq + _perm(sq, swap1)
                d2 = s1 + _perm(s1, swap2) + 1e-12
                dist = d2 * _rsqrt_newton(d2)
                obuf[sl] = jnp.where(m3, dist, diff)
                return carry2

            lax.fori_loop(0, _SB * 4 // 16, compute_body, 0)
            pltpu.async_copy(
                obuf, out.at[pl.ds((wid * nsb + b) * _SB * 4, _SB * 4)], semo)

        fire(0, 0)

        def two_blocks(bb_, carry):
            b0 = 2 * bb_
            b1 = 2 * bb_ + 1
            wait_gathers(0)
            fire(b1, 1)
            compute_write(b0, 0)
            wait_gathers(1)

            @pl.when(b1 + 1 < nsb)
            def _next():
                fire(b1 + 1, 0)

            compute_write(b1, 1)
            return carry

        lax.fori_loop(0, nsb // 2, two_blocks, 0)

        pltpu.make_async_copy(out.at[pl.ds(0, _SB * 4)], obuf0, semo0).wait()
        pltpu.make_async_copy(out.at[pl.ds(0, _SB * 4)], obuf1, semo1).wait()

    return ek


def _node_body(pos_ref, g_ref, b_ref, out_ref):
    p = pos_ref[...]
    v = lax.dot_general(p, g_ref[...], (((1,), (0,)), ((), ())),
                        precision=lax.Precision.HIGHEST,
                        preferred_element_type=jnp.float32)
    out_ref[...] = jnp.clip((v - b_ref[...]) * _INV_R, -1.0, 1.0)


@functools.lru_cache(maxsize=None)
def _node_kernel(n):
    nb = 10000
    assert n % nb == 0
    return pl.pallas_call(
        _node_body,
        grid=(n // nb,),
        in_specs=[
            pl.BlockSpec((nb, _ENV), lambda i: (i, 0)),
            pl.BlockSpec((_ENV, _NF), lambda i: (0, 0)),
            pl.BlockSpec((1, _NF), lambda i: (0, 0)),
        ],
        out_specs=pl.BlockSpec((nb, 2 * _ENV + _HIST * _ENV), lambda i: (i, 0)),
        out_shape=jax.ShapeDtypeStruct((n, 2 * _ENV + _HIST * _ENV), jnp.float32),
    )


def kernel(pos, edge_index):
    n = pos.shape[0]
    m = edge_index.shape[1]
    chunk = _NW * _SB * 2
    n_pad = ((m + chunk - 1) // chunk) * chunk

    posf = pos.astype(jnp.float32)
    pos4 = jnp.pad(posf, ((0, 0), (0, 1))).reshape(-1)
    srcf = jnp.pad(edge_index[0], (0, n_pad - m))
    dstf = jnp.pad(edge_index[1], (0, n_pad - m))

    ef_flat = _edge_kernel(n, n_pad)(pos4, srcf, dstf)
    edge_feature = ef_flat.reshape(n_pad, 4)[:m]
    node_feature = _node_kernel(n)(
        posf, jnp.asarray(_NODE_G), jnp.asarray(_NODE_B))
    return node_feature, edge_feature
